# baseline (speedup 1.0000x reference)
import jax
import jax.numpy as jnp
from jax.experimental import pallas as pl

_BIGF = 1e30


def _dot(a, b):
    return jax.lax.dot_general(a, b, (((1,), (0,)), ((), ())),
                               preferred_element_type=jnp.float32)


def _split2(b):
    hi = b.astype(jnp.bfloat16)
    lo = (b - hi.astype(jnp.float32)).astype(jnp.bfloat16)
    return jnp.concatenate([hi, lo], axis=1)


def _oh_gather(oh16, hilo):
    dn = (((1,), (0,)), ((), ()))
    g = jax.lax.dot_general(oh16, hilo, dn,
                            preferred_element_type=jnp.float32)
    f = hilo.shape[1] // 2
    return g[:, :f] + g[:, f:]


def _fps_coords(X, Y, Z, npoint):
    b, n = X.shape
    lane = jax.lax.broadcasted_iota(jnp.int32, (b, n), 1)
    out_lane = jax.lax.broadcasted_iota(jnp.int32, (b, npoint), 1)
    lx = X[:, 0:1]
    ly = Y[:, 0:1]
    lz = Z[:, 0:1]
    accx = jnp.where(out_lane == 0, lx, 0.0)
    accy = jnp.where(out_lane == 0, ly, 0.0)
    accz = jnp.where(out_lane == 0, lz, 0.0)
    dists = jnp.full((b, n), 1e10, dtype=jnp.float32)

    def body(i, c):
        dists, accx, accy, accz, lx, ly, lz = c
        d = (X - lx) ** 2 + (Y - ly) ** 2 + (Z - lz) ** 2
        dists = jnp.minimum(dists, d)
        nxt = jnp.argmax(dists, axis=1).astype(jnp.int32)[:, None]
        hit = lane == nxt
        lx = jnp.sum(jnp.where(hit, X, 0.0), axis=1, keepdims=True)
        ly = jnp.sum(jnp.where(hit, Y, 0.0), axis=1, keepdims=True)
        lz = jnp.sum(jnp.where(hit, Z, 0.0), axis=1, keepdims=True)
        pick = out_lane == (i + 1)
        accx = jnp.where(pick, lx, accx)
        accy = jnp.where(pick, ly, accy)
        accz = jnp.where(pick, lz, accz)
        return (dists, accx, accy, accz, lx, ly, lz)

    c = jax.lax.fori_loop(0, npoint - 1, body,
                          (dists, accx, accy, accz, lx, ly, lz),
                          unroll=4)
    return c[1], c[2], c[3]


def _fps_kernel(x0, y0, z0,
                x1, y1, z1, x2, y2, z2, x3, y3, z3):
    a, b, c = _fps_coords(x0[...], y0[...], z0[...], 256)
    x1[...], y1[...], z1[...] = a, b, c
    a, b, c = _fps_coords(a, b, c, 128)
    x2[...], y2[...], z2[...] = a, b, c
    a, b, c = _fps_coords(a, b, c, 64)
    x3[...], y3[...], z3[...] = a, b, c


def _dist2(qmat, kx, ky, kz):
    qq = jnp.sum(qmat * qmat, axis=1, keepdims=True)
    kk = kx * kx + ky * ky + kz * kz
    kmat = jnp.concatenate([kx, ky, kz], axis=0)
    qk = _dot(qmat, kmat)
    return jnp.maximum(qq + kk - 2.0 * qk, 0.0)


def _knn_maxpool(D, H, k):
    m_rows, n = D.shape
    lane = jax.lax.broadcasted_iota(jnp.int32, (m_rows, n), 1)
    acc = jnp.zeros((m_rows, H.shape[1]), jnp.float32)
    hilo = _split2(H)
    idx0 = jnp.argmin(D, axis=1).astype(jnp.int32)[:, None]

    def body(j, c):
        D, acc, idx = c
        hit = lane == idx
        oh = jnp.where(hit, 1.0, 0.0).astype(jnp.bfloat16)
        g = _oh_gather(oh, hilo)
        acc = jnp.maximum(acc, g)
        D = jnp.where(hit, _BIGF, D)
        idx = jnp.argmin(D, axis=1).astype(jnp.int32)[:, None]
        return (D, acc, idx)

    D, acc, idx = jax.lax.fori_loop(0, k - 1, body, (D, acc, idx0),
                                    unroll=4)
    oh = jnp.where(lane == idx, 1.0, 0.0).astype(jnp.bfloat16)
    return jnp.maximum(acc, _oh_gather(oh, hilo))


def _interp3(D, F):
    m_rows, n = D.shape
    lane = jax.lax.broadcasted_iota(jnp.int32, (m_rows, n), 1)
    interp = jnp.zeros((m_rows, F.shape[1]), jnp.float32)
    wsum = jnp.zeros((m_rows, 1), jnp.float32)
    hilo = _split2(F)
    for j in range(3):
        idx = jnp.argmin(D, axis=1).astype(jnp.int32)[:, None]
        hit = lane == idx
        m = jnp.sum(jnp.where(hit, D, 0.0), axis=1, keepdims=True)
        w = 1.0 / (jnp.maximum(m, 0.0) + 1e-8)
        g = _oh_gather(jnp.where(hit, 1.0, 0.0).astype(jnp.bfloat16), hilo)
        interp = interp + w * g
        wsum = wsum + w
        if j < 2:
            D = jnp.where(hit, _BIGF, D)
    return interp / wsum


def _relu(x):
    return jnp.maximum(x, 0.0)


def _main_kernel(xyz, x0, y0, z0, msg,
                 p1, x1, y1, z1, p2, x2, y2, z2, p3, x3, y3, z3,
                 c0W1a, c0W1b, c0b1, c0W2, c0b2,
                 c1W1, c1b1, c1W2, c1b2,
                 c2W1, c2b1, c2W2, c2b2,
                 d0Wa, d0Wb, d0b, d1Wa, d1Wb, d1b,
                 d2Wa, d2Wb, d2Wc, d2b,
                 out):
    xyz_m = xyz[0]
    msg_r = msg[0]

    h = _relu(_dot(xyz_m, c0W1a[...]) + _dot(msg_r, c0W1b[...]) + c0b1[...])
    H0 = _relu(_dot(h, c0W2[...]) + c0b2[...])
    D = _dist2(p1[0], x0[0], y0[0], z0[0])
    feat1 = _knn_maxpool(D, H0, 32)

    h = _relu(_dot(feat1, c1W1[...]) + c1b1[...])
    H1 = _relu(_dot(h, c1W2[...]) + c1b2[...])
    D = _dist2(p2[0], x1[0], y1[0], z1[0])
    feat2 = _knn_maxpool(D, H1, 32)

    h = _relu(_dot(feat2, c2W1[...]) + c2b1[...])
    H2 = _relu(_dot(h, c2W2[...]) + c2b2[...])
    D = _dist2(p3[0], x2[0], y2[0], z2[0])
    feat3 = _knn_maxpool(D, H2, 32)

    D = _dist2(p2[0], x3[0], y3[0], z3[0])
    it = _interp3(D, feat3)
    feat2 = _relu(_dot(feat2, d0Wa[...]) + _dot(it, d0Wb[...]) + d0b[...])

    D = _dist2(p1[0], x2[0], y2[0], z2[0])
    it = _interp3(D, feat2)
    feat1 = _relu(_dot(feat1, d1Wa[...]) + _dot(it, d1Wb[...]) + d1b[...])

    D = _dist2(xyz_m, x1[0], y1[0], z1[0])
    it = _interp3(D, feat1)
    res = (_dot(xyz_m, d2Wa[...]) + _dot(msg_r, d2Wb[...])
           + _dot(it, d2Wc[...]) + d2b[...])
    out[0] = res


def kernel(xyz, msg, c0_W1, c0_b1, c0_W2, c0_b2, c1_W1, c1_b1, c1_W2, c1_b2,
           c2_W1, c2_b1, c2_W2, c2_b2, d0_W, d0_b, d1_W, d1_b, d2_W, d2_b):
    B, N, _ = xyz.shape
    f32 = jnp.float32
    x0 = xyz[:, :, 0]
    y0 = xyz[:, :, 1]
    z0 = xyz[:, :, 2]

    fps_shapes = []
    for np_ in (256, 128, 64):
        fps_shapes += [jax.ShapeDtypeStruct((B, np_), f32)] * 3
    x1, y1, z1, x2, y2, z2, x3, y3, z3 = pl.pallas_call(
        _fps_kernel, out_shape=fps_shapes)(x0, y0, z0)

    p1 = jnp.stack([x1, y1, z1], axis=-1)
    p2 = jnp.stack([x2, y2, z2], axis=-1)
    p3 = jnp.stack([x3, y3, z3], axis=-1)

    def r3(a):
        return a[:, None, :]

    per_batch_3d = [xyz, r3(x0), r3(y0), r3(z0), msg[:, None, :],
                    p1, r3(x1), r3(y1), r3(z1),
                    p2, r3(x2), r3(y2), r3(z2),
                    p3, r3(x3), r3(y3), r3(z3)]
    weights = [c0_W1[:3], c0_W1[3:], c0_b1[None, :], c0_W2, c0_b2[None, :],
               c1_W1, c1_b1[None, :], c1_W2, c1_b2[None, :],
               c2_W1, c2_b1[None, :], c2_W2, c2_b2[None, :],
               d0_W[:64], d0_W[64:], d0_b[None, :],
               d1_W[:64], d1_W[64:], d1_b[None, :],
               d2_W[:3], d2_W[3:19], d2_W[19:], d2_b[None, :]]

    def batch_spec(a):
        s = a.shape
        return pl.BlockSpec((1,) + s[1:], lambda b: (b, 0, 0))

    def full_spec(a):
        return pl.BlockSpec(a.shape, lambda b: (0,) * a.ndim)

    in_specs = [batch_spec(a) for a in per_batch_3d] + \
               [full_spec(w) for w in weights]
    out = pl.pallas_call(
        _main_kernel,
        grid=(B,),
        in_specs=in_specs,
        out_specs=pl.BlockSpec((1, N, 3), lambda b: (b, 0, 0)),
        out_shape=jax.ShapeDtypeStruct((B, N, 3), f32),
    )(*per_batch_3d, *weights)
    return out


# speedup vs baseline: 1.0047x; 1.0047x over previous
import jax
import jax.numpy as jnp
from jax.experimental import pallas as pl

_BIGF = 1e30


def _dot(a, b):
    return jax.lax.dot_general(a, b, (((1,), (0,)), ((), ())),
                               preferred_element_type=jnp.float32)


def _split2(b):
    hi = b.astype(jnp.bfloat16)
    lo = (b - hi.astype(jnp.float32)).astype(jnp.bfloat16)
    return jnp.concatenate([hi, lo], axis=1)


def _oh_gather(oh16, hilo):
    dn = (((1,), (0,)), ((), ()))
    g = jax.lax.dot_general(oh16, hilo, dn,
                            preferred_element_type=jnp.float32)
    f = hilo.shape[1] // 2
    return g[:, :f] + g[:, f:]


def _fps_coords(X, Y, Z, npoint):
    b, n = X.shape
    lane = jax.lax.broadcasted_iota(jnp.int32, (b, n), 1)
    out_lane = jax.lax.broadcasted_iota(jnp.int32, (b, npoint), 1)
    lx = X[:, 0:1]
    ly = Y[:, 0:1]
    lz = Z[:, 0:1]
    accx = jnp.where(out_lane == 0, lx, 0.0)
    accy = jnp.where(out_lane == 0, ly, 0.0)
    accz = jnp.where(out_lane == 0, lz, 0.0)
    dists = jnp.full((b, n), 1e10, dtype=jnp.float32)

    def body(i, c):
        dists, accx, accy, accz, lx, ly, lz = c
        d = (X - lx) ** 2 + (Y - ly) ** 2 + (Z - lz) ** 2
        dists = jnp.minimum(dists, d)
        nxt = jnp.argmax(dists, axis=1).astype(jnp.int32)[:, None]
        hit = lane == nxt
        lx = jnp.sum(jnp.where(hit, X, 0.0), axis=1, keepdims=True)
        ly = jnp.sum(jnp.where(hit, Y, 0.0), axis=1, keepdims=True)
        lz = jnp.sum(jnp.where(hit, Z, 0.0), axis=1, keepdims=True)
        pick = out_lane == (i + 1)
        accx = jnp.where(pick, lx, accx)
        accy = jnp.where(pick, ly, accy)
        accz = jnp.where(pick, lz, accz)
        return (dists, accx, accy, accz, lx, ly, lz)

    c = jax.lax.fori_loop(0, npoint - 1, body,
                          (dists, accx, accy, accz, lx, ly, lz),
                          unroll=8)
    return c[1], c[2], c[3]


def _fps_kernel(x0, y0, z0,
                x1, y1, z1, x2, y2, z2, x3, y3, z3):
    a, b, c = _fps_coords(x0[...], y0[...], z0[...], 256)
    x1[...], y1[...], z1[...] = a, b, c
    a, b, c = _fps_coords(a, b, c, 128)
    x2[...], y2[...], z2[...] = a, b, c
    a, b, c = _fps_coords(a, b, c, 64)
    x3[...], y3[...], z3[...] = a, b, c


def _dist2(qmat, kx, ky, kz):
    qq = jnp.sum(qmat * qmat, axis=1, keepdims=True)
    kk = kx * kx + ky * ky + kz * kz
    kmat = jnp.concatenate([kx, ky, kz], axis=0)
    qk = _dot(qmat, kmat)
    return jnp.maximum(qq + kk - 2.0 * qk, 0.0)


def _knn_maxpool(D, H, k):
    m_rows, n = D.shape
    lane = jax.lax.broadcasted_iota(jnp.int32, (m_rows, n), 1)
    acc = jnp.zeros((m_rows, H.shape[1]), jnp.float32)
    hilo = _split2(H)
    idx0 = jnp.argmin(D, axis=1).astype(jnp.int32)[:, None]

    def body(j, c):
        D, acc, idx = c
        hit = lane == idx
        oh = jnp.where(hit, 1.0, 0.0).astype(jnp.bfloat16)
        g = _oh_gather(oh, hilo)
        acc = jnp.maximum(acc, g)
        D = jnp.where(hit, _BIGF, D)
        idx = jnp.argmin(D, axis=1).astype(jnp.int32)[:, None]
        return (D, acc, idx)

    D, acc, idx = jax.lax.fori_loop(0, k - 1, body, (D, acc, idx0),
                                    unroll=4)
    oh = jnp.where(lane == idx, 1.0, 0.0).astype(jnp.bfloat16)
    return jnp.maximum(acc, _oh_gather(oh, hilo))


def _interp3(D, F):
    m_rows, n = D.shape
    lane = jax.lax.broadcasted_iota(jnp.int32, (m_rows, n), 1)
    interp = jnp.zeros((m_rows, F.shape[1]), jnp.float32)
    wsum = jnp.zeros((m_rows, 1), jnp.float32)
    hilo = _split2(F)
    for j in range(3):
        idx = jnp.argmin(D, axis=1).astype(jnp.int32)[:, None]
        hit = lane == idx
        m = jnp.sum(jnp.where(hit, D, 0.0), axis=1, keepdims=True)
        w = 1.0 / (jnp.maximum(m, 0.0) + 1e-8)
        g = _oh_gather(jnp.where(hit, 1.0, 0.0).astype(jnp.bfloat16), hilo)
        interp = interp + w * g
        wsum = wsum + w
        if j < 2:
            D = jnp.where(hit, _BIGF, D)
    return interp / wsum


def _relu(x):
    return jnp.maximum(x, 0.0)


def _main_kernel(xyz, x0, y0, z0, msg,
                 p1, x1, y1, z1, p2, x2, y2, z2, p3, x3, y3, z3,
                 c0W1a, c0W1b, c0b1, c0W2, c0b2,
                 c1W1, c1b1, c1W2, c1b2,
                 c2W1, c2b1, c2W2, c2b2,
                 d0Wa, d0Wb, d0b, d1Wa, d1Wb, d1b,
                 d2Wa, d2Wb, d2Wc, d2b,
                 out):
    xyz_m = xyz[0]
    msg_r = msg[0]

    h = _relu(_dot(xyz_m, c0W1a[...]) + _dot(msg_r, c0W1b[...]) + c0b1[...])
    H0 = _relu(_dot(h, c0W2[...]) + c0b2[...])
    D = _dist2(p1[0], x0[0], y0[0], z0[0])
    feat1 = _knn_maxpool(D, H0, 32)

    h = _relu(_dot(feat1, c1W1[...]) + c1b1[...])
    H1 = _relu(_dot(h, c1W2[...]) + c1b2[...])
    D = _dist2(p2[0], x1[0], y1[0], z1[0])
    feat2 = _knn_maxpool(D, H1, 32)

    h = _relu(_dot(feat2, c2W1[...]) + c2b1[...])
    H2 = _relu(_dot(h, c2W2[...]) + c2b2[...])
    D = _dist2(p3[0], x2[0], y2[0], z2[0])
    feat3 = _knn_maxpool(D, H2, 32)

    D = _dist2(p2[0], x3[0], y3[0], z3[0])
    it = _interp3(D, feat3)
    feat2 = _relu(_dot(feat2, d0Wa[...]) + _dot(it, d0Wb[...]) + d0b[...])

    D = _dist2(p1[0], x2[0], y2[0], z2[0])
    it = _interp3(D, feat2)
    feat1 = _relu(_dot(feat1, d1Wa[...]) + _dot(it, d1Wb[...]) + d1b[...])

    D = _dist2(xyz_m, x1[0], y1[0], z1[0])
    it = _interp3(D, feat1)
    res = (_dot(xyz_m, d2Wa[...]) + _dot(msg_r, d2Wb[...])
           + _dot(it, d2Wc[...]) + d2b[...])
    out[0] = res


def kernel(xyz, msg, c0_W1, c0_b1, c0_W2, c0_b2, c1_W1, c1_b1, c1_W2, c1_b2,
           c2_W1, c2_b1, c2_W2, c2_b2, d0_W, d0_b, d1_W, d1_b, d2_W, d2_b):
    B, N, _ = xyz.shape
    f32 = jnp.float32
    x0 = xyz[:, :, 0]
    y0 = xyz[:, :, 1]
    z0 = xyz[:, :, 2]

    fps_shapes = []
    for np_ in (256, 128, 64):
        fps_shapes += [jax.ShapeDtypeStruct((B, np_), f32)] * 3
    x1, y1, z1, x2, y2, z2, x3, y3, z3 = pl.pallas_call(
        _fps_kernel, out_shape=fps_shapes)(x0, y0, z0)

    p1 = jnp.stack([x1, y1, z1], axis=-1)
    p2 = jnp.stack([x2, y2, z2], axis=-1)
    p3 = jnp.stack([x3, y3, z3], axis=-1)

    def r3(a):
        return a[:, None, :]

    per_batch_3d = [xyz, r3(x0), r3(y0), r3(z0), msg[:, None, :],
                    p1, r3(x1), r3(y1), r3(z1),
                    p2, r3(x2), r3(y2), r3(z2),
                    p3, r3(x3), r3(y3), r3(z3)]
    weights = [c0_W1[:3], c0_W1[3:], c0_b1[None, :], c0_W2, c0_b2[None, :],
               c1_W1, c1_b1[None, :], c1_W2, c1_b2[None, :],
               c2_W1, c2_b1[None, :], c2_W2, c2_b2[None, :],
               d0_W[:64], d0_W[64:], d0_b[None, :],
               d1_W[:64], d1_W[64:], d1_b[None, :],
               d2_W[:3], d2_W[3:19], d2_W[19:], d2_b[None, :]]

    def batch_spec(a):
        s = a.shape
        return pl.BlockSpec((1,) + s[1:], lambda b: (b, 0, 0))

    def full_spec(a):
        return pl.BlockSpec(a.shape, lambda b: (0,) * a.ndim)

    in_specs = [batch_spec(a) for a in per_batch_3d] + \
               [full_spec(w) for w in weights]
    out = pl.pallas_call(
        _main_kernel,
        grid=(B,),
        in_specs=in_specs,
        out_specs=pl.BlockSpec((1, N, 3), lambda b: (b, 0, 0)),
        out_shape=jax.ShapeDtypeStruct((B, N, 3), f32),
    )(*per_batch_3d, *weights)
    return out
